# baseline (speedup 1.0000x reference)
import functools

import jax
import jax.numpy as jnp
from jax import lax
from jax.experimental import pallas as pl
from jax.experimental.pallas import tpu as pltpu
from jax.experimental.pallas import tpu_sc as plsc

N_NODES = 10000
N_EDGES = 320000
D_FEAT = 128
HIDDEN = 128
OUT = 64
BATCH = 4096
N_SAMPLES = 10

NC = 2
NS = 16
NW = NC * NS
CHUNK = 128
CH = 80
E_PAD = NW * CH * CHUNK
N_PAD = 10240
STRIPE = N_PAD // NS


@functools.cache
def _mesh():
    return plsc.VectorSubcoreMesh(
        core_axis_name="c", subcore_axis_name="s", num_cores=NC, num_subcores=NS
    )


def _relu(v):
    return jnp.maximum(v, 0.0)


def _tc1_body(x_ref, ws_ref, wn_ref, b_ref, s1_ref, y1_ref):
    xb = x_ref[...]
    s1_ref[...] = (
        jnp.dot(xb, ws_ref[...], preferred_element_type=jnp.float32) + b_ref[...]
    )
    y1_ref[...] = jnp.dot(xb, wn_ref[...], preferred_element_type=jnp.float32)


def _tc1(x, w_self1, w_neigh1, b1):
    r = 1000
    grid = N_NODES // r
    return pl.pallas_call(
        _tc1_body,
        grid=(grid,),
        in_specs=[
            pl.BlockSpec((r, D_FEAT), lambda i: (i, 0)),
            pl.BlockSpec((D_FEAT, HIDDEN), lambda i: (0, 0)),
            pl.BlockSpec((D_FEAT, HIDDEN), lambda i: (0, 0)),
            pl.BlockSpec((1, HIDDEN), lambda i: (0, 0)),
        ],
        out_specs=[
            pl.BlockSpec((r, HIDDEN), lambda i: (i, 0)),
            pl.BlockSpec((r, HIDDEN), lambda i: (i, 0)),
        ],
        out_shape=[
            jax.ShapeDtypeStruct((N_NODES, HIDDEN), jnp.float32),
            jax.ShapeDtypeStruct((N_NODES, HIDDEN), jnp.float32),
        ],
    )(x, w_self1, w_neigh1, b1.reshape(1, HIDDEN))


def _tc2_body(s1_ref, aa_ref, ab_ref, da_ref, db_ref, ws_ref, wn_ref, b_ref,
              s2_ref, y2_ref):
    deg = da_ref[...][:, 0:1] + db_ref[...][:, 0:1]
    invd = 1.0 / jnp.maximum(deg, 1.0)
    h = _relu(s1_ref[...] + (aa_ref[...] + ab_ref[...]) * invd)
    s2 = jnp.dot(h, ws_ref[...], preferred_element_type=jnp.float32) + b_ref[...]
    y2 = jnp.dot(h, wn_ref[...], preferred_element_type=jnp.float32)
    s2_ref[...] = s2
    y2_ref[...] = jnp.concatenate([y2, jnp.zeros_like(y2)], axis=1)


def _tc2(s1, agg1a, agg1b, dga, dgb, w_self2, w_neigh2, b2):
    r = 1000
    grid = N_NODES // r
    return pl.pallas_call(
        _tc2_body,
        grid=(grid,),
        in_specs=[
            pl.BlockSpec((r, HIDDEN), lambda i: (i, 0)),
            pl.BlockSpec((r, HIDDEN), lambda i: (i, 0)),
            pl.BlockSpec((r, HIDDEN), lambda i: (i, 0)),
            pl.BlockSpec((r, 16), lambda i: (i, 0)),
            pl.BlockSpec((r, 16), lambda i: (i, 0)),
            pl.BlockSpec((HIDDEN, OUT), lambda i: (0, 0)),
            pl.BlockSpec((HIDDEN, OUT), lambda i: (0, 0)),
            pl.BlockSpec((1, OUT), lambda i: (0, 0)),
        ],
        out_specs=[
            pl.BlockSpec((r, OUT), lambda i: (i, 0)),
            pl.BlockSpec((r, 2 * OUT), lambda i: (i, 0)),
        ],
        out_shape=[
            jax.ShapeDtypeStruct((N_NODES, OUT), jnp.float32),
            jax.ShapeDtypeStruct((N_NODES, 2 * OUT), jnp.float32),
        ],
    )(s1, agg1a, agg1b, dga, dgb, w_self2, w_neigh2, b2.reshape(1, OUT))


def _tc3_body(s2_ref, aa_ref, ab_ref, da_ref, db_ref, emb_ref):
    deg = da_ref[...][:, 0:1] + db_ref[...][:, 0:1]
    invd = 1.0 / jnp.maximum(deg, 1.0)
    agg = (aa_ref[...] + ab_ref[...])[:, :OUT]
    emb = s2_ref[...] + agg * invd
    emb_ref[...] = jnp.concatenate([emb, jnp.zeros_like(emb)], axis=1)


def _tc3(s2, agg2a, agg2b, dga, dgb):
    r = 1000
    grid = N_NODES // r
    return pl.pallas_call(
        _tc3_body,
        grid=(grid,),
        in_specs=[
            pl.BlockSpec((r, OUT), lambda i: (i, 0)),
            pl.BlockSpec((r, 2 * OUT), lambda i: (i, 0)),
            pl.BlockSpec((r, 2 * OUT), lambda i: (i, 0)),
            pl.BlockSpec((r, 16), lambda i: (i, 0)),
            pl.BlockSpec((r, 16), lambda i: (i, 0)),
        ],
        out_specs=pl.BlockSpec((r, 2 * OUT), lambda i: (i, 0)),
        out_shape=jax.ShapeDtypeStruct((N_NODES, 2 * OUT), jnp.float32),
    )(s2, agg2a, agg2b, dga, dgb)


def _tc4_body(rated_ref, item_ref, pred_ref):
    r3 = rated_ref[...]
    t = item_ref[...]
    t3 = t[:, None, :]
    dot = jnp.sum(r3 * t3, axis=2)
    nr = jnp.sqrt(jnp.sum(r3 * r3, axis=2))
    ni = jnp.sqrt(jnp.sum(t * t, axis=1))[:, None]
    sim = dot / jnp.maximum(nr * ni, 1e-6)
    pred = jnp.sum(sim, axis=1)
    pred_ref[...] = jnp.broadcast_to(pred[:, None], pred_ref.shape)


def _tc4(rated3, item_emb):
    rb = 512
    grid = BATCH // rb
    pred2d = pl.pallas_call(
        _tc4_body,
        grid=(grid,),
        in_specs=[
            pl.BlockSpec((rb, N_SAMPLES, 2 * OUT), lambda i: (i, 0, 0)),
            pl.BlockSpec((rb, 2 * OUT), lambda i: (i, 0)),
        ],
        out_specs=pl.BlockSpec((rb, 8), lambda i: (i, 0)),
        out_shape=jax.ShapeDtypeStruct((BATCH, 8), jnp.float32),
    )(rated3, item_emb)
    return pred2d[:, 0]


G = 8
PAIR = 2 * G
NG = CH // G
NGP = CH // PAIR


def _edge_pass(y_tab, agg_sp, srcg_hbm, wng, dstv, slots, bufs, gsems, ssems,
               isems):
    def _gather_start(slot, row, buf, sem):
        for h in range(2):
            pltpu.async_copy(
                y_tab.at[slot.at[row, pl.ds(h * 64, 64)]],
                buf.at[pl.ds(h * 64, 64)], sem,
            )

    def _gather_wait(slot, row, buf, sem):
        for h in range(2):
            pltpu.make_async_copy(
                y_tab.at[slot.at[row, pl.ds(h * 64, 64)]],
                buf.at[pl.ds(h * 64, 64)], sem,
            ).wait()

    def _scatter_start(k, buf, sem):
        for h in range(2):
            pltpu.async_copy(
                buf.at[pl.ds(h * 64, 64)],
                agg_sp.at[dstv.at[k, pl.ds(h * 64, 64)]], sem, add=True,
            )

    def _scatter_wait(k, buf, sem):
        for h in range(2):
            pltpu.make_async_copy(
                buf.at[pl.ds(h * 64, 64)],
                agg_sp.at[dstv.at[k, pl.ds(h * 64, 64)]], sem,
            ).wait()

    pltpu.sync_copy(srcg_hbm.at[wng], slots[0])
    pltpu.sync_copy(srcg_hbm.at[wng + 1], slots[1])
    _gather_start(slots[0], 0, bufs[0], gsems[0])

    def outer(p, carry):
        k0 = p * PAIR
        for j in range(PAIR):
            k = k0 + j
            b = j % 2
            nb = 1 - b
            slot = slots[(j // G) % 2]
            row = j % G
            _gather_wait(slot, row, bufs[b], gsems[b])
            _scatter_start(k, bufs[b], ssems[b])
            if j == G - 1:
                @pl.when(p < NGP - 1)
                def _():
                    pltpu.async_copy(
                        srcg_hbm.at[wng + 2 * p + 2], slots[0], isems[0]
                    )
            if j == PAIR - 1:
                @pl.when(p < NGP - 1)
                def _():
                    _scatter_wait(k - 1, bufs[nb], ssems[nb])
                    pltpu.async_copy(
                        srcg_hbm.at[wng + 2 * p + 3], slots[1], isems[1]
                    )
                    pltpu.make_async_copy(
                        srcg_hbm.at[wng + 2 * p + 2], slots[0], isems[0]
                    ).wait()
                    _gather_start(slots[0], 0, bufs[nb], gsems[nb])
            else:
                if j == 0:
                    @pl.when(p > 0)
                    def _():
                        _scatter_wait(k - 1, bufs[nb], ssems[nb])
                else:
                    _scatter_wait(k - 1, bufs[nb], ssems[nb])
                if j == G - 1:
                    @pl.when(p >= 1)
                    def _():
                        pltpu.make_async_copy(
                            srcg_hbm.at[wng + 2 * p + 1], slots[1], isems[1]
                        ).wait()
                nslot = slots[((j + 1) // G) % 2]
                nrow = (j + 1) % G
                _gather_start(nslot, nrow, bufs[nb], gsems[nb])
        return carry

    lax.fori_loop(0, NGP, outer, 0)
    _scatter_wait(CH - 2, bufs[0], ssems[0])
    _scatter_wait(CH - 1, bufs[1], ssems[1])


M16 = CH * CHUNK // 16


def _sc_deg_body(dstf_hbm, deg_out, dstv, tbl, cbuf, outbuf, stage_sp):
    c = lax.axis_index("c")
    s = lax.axis_index("s")
    w = c * NS + s
    pltpu.sync_copy(dstf_hbm.at[w], dstv)

    def zero(j, carry):
        tbl[pl.ds(j * 16, 16)] = jnp.zeros((16,), jnp.float32)
        return carry

    lax.fori_loop(0, N_PAD // 16, zero, 0)

    ones = jnp.ones((16,), jnp.float32)

    def acc(j, carry):
        iv = dstv[pl.ds(j * 16, 16)]
        plsc.addupdate_scatter(tbl, [iv], ones)
        return carry

    lax.fori_loop(0, M16, acc, 0)

    pltpu.sync_copy(tbl, stage_sp.at[s])
    plsc.subcore_barrier()
    row0 = s * STRIPE
    pltpu.sync_copy(stage_sp.at[:, pl.ds(row0, STRIPE)], cbuf)

    def win(v, carry):
        base = v * 16
        d = cbuf[0, pl.ds(base, 16)]
        for t in range(1, NS):
            d = d + cbuf[t, pl.ds(base, 16)]
        for i in range(16):
            outbuf[base + i, :] = jnp.full((16,), d[i], jnp.float32)
        return carry

    lax.fori_loop(0, STRIPE // 16, win, 0)
    pltpu.sync_copy(outbuf, deg_out.at[c, pl.ds(row0, STRIPE)])


def _sc_deg(dstf):
    f = functools.partial(
        pl.kernel,
        out_type=jax.ShapeDtypeStruct((NC, N_PAD, 16), jnp.float32),
        mesh=_mesh(),
        compiler_params=pltpu.CompilerParams(needs_layout_passes=False),
        scratch_types=[
            pltpu.VMEM((CH * CHUNK,), jnp.int32),
            pltpu.VMEM((N_PAD,), jnp.float32),
            pltpu.VMEM((NS, STRIPE), jnp.float32),
            pltpu.VMEM((STRIPE, 16), jnp.float32),
            pltpu.VMEM_SHARED((NS, N_PAD), jnp.float32),
        ],
    )(_sc_deg_body)
    return f(dstf)


def _sc_layer_body(y_tab, srcg_hbm, dst_hbm, z128,
                   agg_out,
                   dstv, slot0, slot1, buf0, buf1, agg_sp,
                   gsem0, gsem1, ssem0, ssem1, isem0, isem1):
    c = lax.axis_index("c")
    s = lax.axis_index("s")
    w = c * NS + s
    wng = w * NG
    row0 = s * STRIPE
    pltpu.sync_copy(z128, agg_sp.at[pl.ds(row0, STRIPE)])
    pltpu.sync_copy(dst_hbm.at[w], dstv)
    plsc.subcore_barrier()

    _edge_pass(y_tab, agg_sp, srcg_hbm, wng, dstv, (slot0, slot1),
               (buf0, buf1), (gsem0, gsem1), (ssem0, ssem1), (isem0, isem1))

    plsc.subcore_barrier()

    pltpu.sync_copy(agg_sp.at[pl.ds(row0, STRIPE)], agg_out.at[c, pl.ds(row0, STRIPE)])


def _sc_layer(y_tab, srcg, dst3, z128):
    f = functools.partial(
        pl.kernel,
        out_type=jax.ShapeDtypeStruct((NC, N_PAD, 128), jnp.float32),
        mesh=_mesh(),
        scratch_types=[
            pltpu.VMEM((CH, CHUNK), jnp.int32),
            pltpu.VMEM((G, CHUNK), jnp.int32),
            pltpu.VMEM((G, CHUNK), jnp.int32),
            pltpu.VMEM((CHUNK, 128), jnp.float32),
            pltpu.VMEM((CHUNK, 128), jnp.float32),
            pltpu.VMEM_SHARED((N_PAD, 128), jnp.float32),
            pltpu.SemaphoreType.DMA,
            pltpu.SemaphoreType.DMA,
            pltpu.SemaphoreType.DMA,
            pltpu.SemaphoreType.DMA,
            pltpu.SemaphoreType.DMA,
            pltpu.SemaphoreType.DMA,
        ],
    )(_sc_layer_body)
    return f(y_tab, srcg, dst3, z128)


_SRC_CH_W = (BATCH * N_SAMPLES) // NW // CHUNK
_ITEM_CH_W = BATCH // NW // CHUNK


def _sc_gather_body(emb, sidx_hbm, iidx_hbm, rated, item_out,
                    sidxv, iidxv, buf0, buf1, gsem0, gsem1, wsem0, wsem1):
    c = lax.axis_index("c")
    s = lax.axis_index("s")
    w = c * NS + s
    pltpu.sync_copy(sidx_hbm.at[w], sidxv)
    pltpu.sync_copy(iidx_hbm.at[w], iidxv)
    bufs = (buf0, buf1)
    gsems = (gsem0, gsem1)
    wsems = (wsem0, wsem1)
    for k in range(_SRC_CH_W):
        b = k % 2
        if k >= 2:
            pltpu.make_async_copy(
                bufs[b],
                rated.at[pl.ds((w * _SRC_CH_W + k - 2) * CHUNK, CHUNK)],
                wsems[b],
            ).wait()
        pltpu.async_copy(emb.at[sidxv.at[k]], bufs[b], gsems[b])
        pltpu.make_async_copy(emb.at[sidxv.at[k]], bufs[b], gsems[b]).wait()
        pltpu.async_copy(
            bufs[b],
            rated.at[pl.ds((w * _SRC_CH_W + k) * CHUNK, CHUNK)],
            wsems[b],
        )
    for k in range(_SRC_CH_W - 2, _SRC_CH_W):
        b = k % 2
        pltpu.make_async_copy(
            bufs[b],
            rated.at[pl.ds((w * _SRC_CH_W + k) * CHUNK, CHUNK)],
            wsems[b],
        ).wait()
    pltpu.async_copy(emb.at[iidxv.at[0]], buf0, gsem0)
    pltpu.make_async_copy(emb.at[iidxv.at[0]], buf0, gsem0).wait()
    pltpu.sync_copy(buf0, item_out.at[pl.ds(w * CHUNK, CHUNK)])


def _sc_gather(emb, sidx3, iidx3):
    f = functools.partial(
        pl.kernel,
        out_type=[
            jax.ShapeDtypeStruct((BATCH * N_SAMPLES, 2 * OUT), jnp.float32),
            jax.ShapeDtypeStruct((BATCH, 2 * OUT), jnp.float32),
        ],
        mesh=_mesh(),
        scratch_types=[
            pltpu.VMEM((_SRC_CH_W, CHUNK), jnp.int32),
            pltpu.VMEM((_ITEM_CH_W, CHUNK), jnp.int32),
            pltpu.VMEM((CHUNK, 2 * OUT), jnp.float32),
            pltpu.VMEM((CHUNK, 2 * OUT), jnp.float32),
            pltpu.SemaphoreType.DMA,
            pltpu.SemaphoreType.DMA,
            pltpu.SemaphoreType.DMA,
            pltpu.SemaphoreType.DMA,
        ],
    )(_sc_gather_body)
    return f(emb, sidx3, iidx3)


def kernel(x, edge_index, source, item_idx, W_self1, W_neigh1, b1,
           W_self2, W_neigh2, b2):
    src = edge_index[0].astype(jnp.int32)
    dst = edge_index[1].astype(jnp.int32)
    pad = E_PAD - N_EDGES
    srcg = jnp.concatenate([src, jnp.zeros((pad,), jnp.int32)]).reshape(
        NW * NG, G, CHUNK
    )
    dst3 = jnp.concatenate([dst, jnp.full((pad,), N_NODES, jnp.int32)]).reshape(
        NW, CH, CHUNK
    )
    z128 = jnp.zeros((STRIPE, 128), jnp.float32)

    dstf = dst3.reshape(NW, CH * CHUNK)
    dg = _sc_deg(dstf)
    dga = dg[0, :N_NODES]
    dgb = dg[1, :N_NODES]

    s1, y1 = _tc1(x, W_self1, W_neigh1, b1)
    agg1 = _sc_layer(y1, srcg, dst3, z128)
    agg1a = agg1[0]
    agg1b = agg1[1]

    s2, y2p = _tc2(
        s1, agg1a[:N_NODES], agg1b[:N_NODES], dga, dgb, W_self2, W_neigh2, b2
    )
    agg2 = _sc_layer(y2p, srcg, dst3, z128)
    agg2a = agg2[0]
    agg2b = agg2[1]
    embp = _tc3(s2, agg2a[:N_NODES], agg2b[:N_NODES], dga, dgb)

    sidx3 = source.astype(jnp.int32).reshape(NW, _SRC_CH_W, CHUNK)
    iidx3 = item_idx.astype(jnp.int32).reshape(NW, _ITEM_CH_W, CHUNK)
    rated, item_emb = _sc_gather(embp, sidx3, iidx3)
    pred = _tc4(rated.reshape(BATCH, N_SAMPLES, 2 * OUT), item_emb)
    return pred


# speedup vs baseline: 1.0950x; 1.0950x over previous
import functools

import jax
import jax.numpy as jnp
from jax import lax
from jax.experimental import pallas as pl
from jax.experimental.pallas import tpu as pltpu
from jax.experimental.pallas import tpu_sc as plsc

N_NODES = 10000
N_EDGES = 320000
D_FEAT = 128
HIDDEN = 128
OUT = 64
BATCH = 4096
N_SAMPLES = 10

NC = 2
NS = 16
NW = NC * NS
CHUNK = 128
CH0 = 120
CH1 = 40
CHT = CH0 + CH1
E_PAD = NS * CHT * CHUNK
N_PAD = 10112
STRIPE = N_PAD // NS
N_PAD_DEG = 10240
STRIPE_DEG = N_PAD_DEG // NS


@functools.cache
def _mesh():
    return plsc.VectorSubcoreMesh(
        core_axis_name="c", subcore_axis_name="s", num_cores=NC, num_subcores=NS
    )


def _relu(v):
    return jnp.maximum(v, 0.0)


def _tc1_body(x_ref, ws_ref, wn_ref, b_ref, s1_ref, y1_ref):
    xb = x_ref[...]
    s1_ref[...] = (
        jnp.dot(xb, ws_ref[...], preferred_element_type=jnp.float32) + b_ref[...]
    )
    y1_ref[...] = jnp.dot(xb, wn_ref[...], preferred_element_type=jnp.float32)


def _tc1(x, w_self1, w_neigh1, b1):
    r = 1000
    grid = N_NODES // r
    return pl.pallas_call(
        _tc1_body,
        grid=(grid,),
        in_specs=[
            pl.BlockSpec((r, D_FEAT), lambda i: (i, 0)),
            pl.BlockSpec((D_FEAT, HIDDEN), lambda i: (0, 0)),
            pl.BlockSpec((D_FEAT, HIDDEN), lambda i: (0, 0)),
            pl.BlockSpec((1, HIDDEN), lambda i: (0, 0)),
        ],
        out_specs=[
            pl.BlockSpec((r, HIDDEN), lambda i: (i, 0)),
            pl.BlockSpec((r, HIDDEN), lambda i: (i, 0)),
        ],
        out_shape=[
            jax.ShapeDtypeStruct((N_NODES, HIDDEN), jnp.float32),
            jax.ShapeDtypeStruct((N_NODES, HIDDEN), jnp.float32),
        ],
    )(x, w_self1, w_neigh1, b1.reshape(1, HIDDEN))


def _tc2_body(s1_ref, aa_ref, ab_ref, da_ref, db_ref, ws_ref, wn_ref, b_ref,
              s2_ref, y2_ref):
    deg = da_ref[...][:, 0:1] + db_ref[...][:, 0:1]
    invd = 1.0 / jnp.maximum(deg, 1.0)
    h = _relu(s1_ref[...] + (aa_ref[...] + ab_ref[...]) * invd)
    s2 = jnp.dot(h, ws_ref[...], preferred_element_type=jnp.float32) + b_ref[...]
    y2 = jnp.dot(h, wn_ref[...], preferred_element_type=jnp.float32)
    s2_ref[...] = s2
    y2_ref[...] = jnp.concatenate([y2, jnp.zeros_like(y2)], axis=1)


def _tc2(s1, agg1a, agg1b, dga, dgb, w_self2, w_neigh2, b2):
    r = 1000
    grid = N_NODES // r
    return pl.pallas_call(
        _tc2_body,
        grid=(grid,),
        in_specs=[
            pl.BlockSpec((r, HIDDEN), lambda i: (i, 0)),
            pl.BlockSpec((r, HIDDEN), lambda i: (i, 0)),
            pl.BlockSpec((r, HIDDEN), lambda i: (i, 0)),
            pl.BlockSpec((r, 16), lambda i: (i, 0)),
            pl.BlockSpec((r, 16), lambda i: (i, 0)),
            pl.BlockSpec((HIDDEN, OUT), lambda i: (0, 0)),
            pl.BlockSpec((HIDDEN, OUT), lambda i: (0, 0)),
            pl.BlockSpec((1, OUT), lambda i: (0, 0)),
        ],
        out_specs=[
            pl.BlockSpec((r, OUT), lambda i: (i, 0)),
            pl.BlockSpec((r, 2 * OUT), lambda i: (i, 0)),
        ],
        out_shape=[
            jax.ShapeDtypeStruct((N_NODES, OUT), jnp.float32),
            jax.ShapeDtypeStruct((N_NODES, 2 * OUT), jnp.float32),
        ],
    )(s1, agg1a, agg1b, dga, dgb, w_self2, w_neigh2, b2.reshape(1, OUT))


def _tc3_body(s2_ref, aa_ref, ab_ref, da_ref, db_ref, emb_ref):
    deg = da_ref[...][:, 0:1] + db_ref[...][:, 0:1]
    invd = 1.0 / jnp.maximum(deg, 1.0)
    agg = (aa_ref[...] + ab_ref[...])[:, :OUT]
    emb = s2_ref[...] + agg * invd
    emb_ref[...] = jnp.concatenate([emb, jnp.zeros_like(emb)], axis=1)


def _tc3(s2, agg2a, agg2b, dga, dgb):
    r = 1000
    grid = N_NODES // r
    return pl.pallas_call(
        _tc3_body,
        grid=(grid,),
        in_specs=[
            pl.BlockSpec((r, OUT), lambda i: (i, 0)),
            pl.BlockSpec((r, 2 * OUT), lambda i: (i, 0)),
            pl.BlockSpec((r, 2 * OUT), lambda i: (i, 0)),
            pl.BlockSpec((r, 16), lambda i: (i, 0)),
            pl.BlockSpec((r, 16), lambda i: (i, 0)),
        ],
        out_specs=pl.BlockSpec((r, 2 * OUT), lambda i: (i, 0)),
        out_shape=jax.ShapeDtypeStruct((N_NODES, 2 * OUT), jnp.float32),
    )(s2, agg2a, agg2b, dga, dgb)


def _tc4_body(rated_ref, item_ref, pred_ref):
    r3 = rated_ref[...]
    t = item_ref[...]
    t3 = t[:, None, :]
    dot = jnp.sum(r3 * t3, axis=2)
    nr = jnp.sqrt(jnp.sum(r3 * r3, axis=2))
    ni = jnp.sqrt(jnp.sum(t * t, axis=1))[:, None]
    sim = dot / jnp.maximum(nr * ni, 1e-6)
    pred = jnp.sum(sim, axis=1)
    pred_ref[...] = jnp.broadcast_to(pred[:, None], pred_ref.shape)


def _tc4(rated3, item_emb):
    rb = 512
    grid = BATCH // rb
    pred2d = pl.pallas_call(
        _tc4_body,
        grid=(grid,),
        in_specs=[
            pl.BlockSpec((rb, N_SAMPLES, 2 * OUT), lambda i: (i, 0, 0)),
            pl.BlockSpec((rb, 2 * OUT), lambda i: (i, 0)),
        ],
        out_specs=pl.BlockSpec((rb, 8), lambda i: (i, 0)),
        out_shape=jax.ShapeDtypeStruct((BATCH, 8), jnp.float32),
    )(rated3, item_emb)
    return pred2d[:, 0]


G = 4
PAIR = 2 * G
NG0 = CH0 // G
NG1 = CH1 // G
NGP0 = CH0 // PAIR
NGP1 = CH1 // PAIR


def _edge_pass(ch, ngp, y_tab, agg_sp, srcg_hbm, gbase, dstv, slots, bufs,
               gsems, ssems, isems):
    def _gather_start(slot, row, buf, sem):
        for h in range(2):
            pltpu.async_copy(
                y_tab.at[slot.at[row, pl.ds(h * 64, 64)]],
                buf.at[pl.ds(h * 64, 64)], sem,
            )

    def _gather_wait(slot, row, buf, sem):
        for h in range(2):
            pltpu.make_async_copy(
                y_tab.at[slot.at[row, pl.ds(h * 64, 64)]],
                buf.at[pl.ds(h * 64, 64)], sem,
            ).wait()

    def _scatter_start(k, buf, sem):
        for h in range(2):
            pltpu.async_copy(
                buf.at[pl.ds(h * 64, 64)],
                agg_sp.at[dstv.at[k, pl.ds(h * 64, 64)]], sem, add=True,
            )

    def _scatter_wait(k, buf, sem):
        for h in range(2):
            pltpu.make_async_copy(
                buf.at[pl.ds(h * 64, 64)],
                agg_sp.at[dstv.at[k, pl.ds(h * 64, 64)]], sem,
            ).wait()

    pltpu.sync_copy(srcg_hbm.at[gbase], slots[0])
    pltpu.sync_copy(srcg_hbm.at[gbase + 1], slots[1])
    _gather_start(slots[0], 0, bufs[0], gsems[0])

    def outer(p, carry):
        k0 = p * PAIR
        for j in range(PAIR):
            k = k0 + j
            b = j % 2
            nb = 1 - b
            slot = slots[(j // G) % 2]
            row = j % G
            _gather_wait(slot, row, bufs[b], gsems[b])
            _scatter_start(k, bufs[b], ssems[b])
            if j == G - 1:
                @pl.when(p < ngp - 1)
                def _():
                    pltpu.async_copy(
                        srcg_hbm.at[gbase + 2 * p + 2], slots[0], isems[0]
                    )
            if j == PAIR - 1:
                @pl.when(p < ngp - 1)
                def _():
                    _scatter_wait(k - 1, bufs[nb], ssems[nb])
                    pltpu.async_copy(
                        srcg_hbm.at[gbase + 2 * p + 3], slots[1], isems[1]
                    )
                    pltpu.make_async_copy(
                        srcg_hbm.at[gbase + 2 * p + 2], slots[0], isems[0]
                    ).wait()
                    _gather_start(slots[0], 0, bufs[nb], gsems[nb])
            else:
                if j == 0:
                    @pl.when(p > 0)
                    def _():
                        _scatter_wait(k - 1, bufs[nb], ssems[nb])
                else:
                    _scatter_wait(k - 1, bufs[nb], ssems[nb])
                if j == G - 1:
                    @pl.when(p >= 1)
                    def _():
                        pltpu.make_async_copy(
                            srcg_hbm.at[gbase + 2 * p + 1], slots[1], isems[1]
                        ).wait()
                nslot = slots[((j + 1) // G) % 2]
                nrow = (j + 1) % G
                _gather_start(nslot, nrow, bufs[nb], gsems[nb])
        return carry

    lax.fori_loop(0, ngp, outer, 0)
    _scatter_wait(ch - 2, bufs[0], ssems[0])
    _scatter_wait(ch - 1, bufs[1], ssems[1])


M16 = E_PAD // NW // 16


def _sc_deg_body(dstf_hbm, deg_out, dstv, tbl, cbuf, outbuf, stage_sp):
    c = lax.axis_index("c")
    s = lax.axis_index("s")
    w = c * NS + s
    pltpu.sync_copy(dstf_hbm.at[w], dstv)

    def zero(j, carry):
        tbl[pl.ds(j * 16, 16)] = jnp.zeros((16,), jnp.float32)
        return carry

    lax.fori_loop(0, N_PAD_DEG // 16, zero, 0)

    ones = jnp.ones((16,), jnp.float32)

    def acc(j, carry):
        iv = dstv[pl.ds(j * 16, 16)]
        plsc.addupdate_scatter(tbl, [iv], ones)
        return carry

    lax.fori_loop(0, M16, acc, 0)

    pltpu.sync_copy(tbl, stage_sp.at[s])
    plsc.subcore_barrier()
    row0 = s * STRIPE_DEG
    pltpu.sync_copy(stage_sp.at[:, pl.ds(row0, STRIPE_DEG)], cbuf)

    def win(v, carry):
        base = v * 16
        d = cbuf[0, pl.ds(base, 16)]
        for t in range(1, NS):
            d = d + cbuf[t, pl.ds(base, 16)]
        for i in range(16):
            outbuf[base + i, :] = jnp.full((16,), d[i], jnp.float32)
        return carry

    lax.fori_loop(0, STRIPE_DEG // 16, win, 0)
    pltpu.sync_copy(outbuf, deg_out.at[c, pl.ds(row0, STRIPE_DEG)])


def _sc_deg(dstf):
    f = functools.partial(
        pl.kernel,
        out_type=jax.ShapeDtypeStruct((NC, N_PAD_DEG, 16), jnp.float32),
        mesh=_mesh(),
        compiler_params=pltpu.CompilerParams(needs_layout_passes=False),
        scratch_types=[
            pltpu.VMEM((E_PAD // NW,), jnp.int32),
            pltpu.VMEM((N_PAD_DEG,), jnp.float32),
            pltpu.VMEM((NS, STRIPE_DEG), jnp.float32),
            pltpu.VMEM((STRIPE_DEG, 16), jnp.float32),
            pltpu.VMEM_SHARED((NS, N_PAD_DEG), jnp.float32),
        ],
    )(_sc_deg_body)
    return f(dstf)


def _sc_layer_body(y_tab, srcg_hbm, dstc_hbm, z128,
                   agg_out,
                   dstv, slot0, slot1, buf0, buf1, agg_sp,
                   gsem0, gsem1, ssem0, ssem1, isem0, isem1):
    c = lax.axis_index("c")
    s = lax.axis_index("s")
    row0 = s * STRIPE
    pltpu.sync_copy(z128, agg_sp.at[pl.ds(row0, STRIPE)])
    plsc.subcore_barrier()

    slots = (slot0, slot1)
    bufs = (buf0, buf1)
    gsems = (gsem0, gsem1)
    ssems = (ssem0, ssem1)
    isems = (isem0, isem1)

    @pl.when(c == 0)
    def _():
        pltpu.sync_copy(dstc_hbm.at[pl.ds(s * CH0, CH0)], dstv.at[pl.ds(0, CH0)])
        _edge_pass(CH0, NGP0, y_tab, agg_sp, srcg_hbm, s * NG0, dstv, slots,
                   bufs, gsems, ssems, isems)

    @pl.when(c == 1)
    def _():
        pltpu.sync_copy(
            dstc_hbm.at[pl.ds(NS * CH0 + s * CH1, CH1)], dstv.at[pl.ds(0, CH1)]
        )
        _edge_pass(CH1, NGP1, y_tab, agg_sp, srcg_hbm, NS * NG0 + s * NG1,
                   dstv, slots, bufs, gsems, ssems, isems)

    plsc.subcore_barrier()

    pltpu.sync_copy(agg_sp.at[pl.ds(row0, STRIPE)], agg_out.at[c, pl.ds(row0, STRIPE)])


def _sc_layer(y_tab, srcg, dstc, z128):
    f = functools.partial(
        pl.kernel,
        out_type=jax.ShapeDtypeStruct((NC, N_PAD, 128), jnp.float32),
        mesh=_mesh(),
        scratch_types=[
            pltpu.VMEM((CH0, CHUNK), jnp.int32),
            pltpu.VMEM((G, CHUNK), jnp.int32),
            pltpu.VMEM((G, CHUNK), jnp.int32),
            pltpu.VMEM((CHUNK, 128), jnp.float32),
            pltpu.VMEM((CHUNK, 128), jnp.float32),
            pltpu.VMEM_SHARED((N_PAD, 128), jnp.float32),
            pltpu.SemaphoreType.DMA,
            pltpu.SemaphoreType.DMA,
            pltpu.SemaphoreType.DMA,
            pltpu.SemaphoreType.DMA,
            pltpu.SemaphoreType.DMA,
            pltpu.SemaphoreType.DMA,
        ],
    )(_sc_layer_body)
    return f(y_tab, srcg, dstc, z128)


_SRC_CH_W = (BATCH * N_SAMPLES) // NW // CHUNK
_ITEM_CH_W = BATCH // NW // CHUNK


def _sc_gather_body(emb, sidx_hbm, iidx_hbm, rated, item_out,
                    sidxv, iidxv, buf0, buf1, gsem0, gsem1, wsem0, wsem1):
    c = lax.axis_index("c")
    s = lax.axis_index("s")
    w = c * NS + s
    pltpu.sync_copy(sidx_hbm.at[w], sidxv)
    pltpu.sync_copy(iidx_hbm.at[w], iidxv)
    bufs = (buf0, buf1)
    gsems = (gsem0, gsem1)
    wsems = (wsem0, wsem1)
    for k in range(_SRC_CH_W):
        b = k % 2
        if k >= 2:
            pltpu.make_async_copy(
                bufs[b],
                rated.at[pl.ds((w * _SRC_CH_W + k - 2) * CHUNK, CHUNK)],
                wsems[b],
            ).wait()
        pltpu.async_copy(emb.at[sidxv.at[k]], bufs[b], gsems[b])
        pltpu.make_async_copy(emb.at[sidxv.at[k]], bufs[b], gsems[b]).wait()
        pltpu.async_copy(
            bufs[b],
            rated.at[pl.ds((w * _SRC_CH_W + k) * CHUNK, CHUNK)],
            wsems[b],
        )
    for k in range(_SRC_CH_W - 2, _SRC_CH_W):
        b = k % 2
        pltpu.make_async_copy(
            bufs[b],
            rated.at[pl.ds((w * _SRC_CH_W + k) * CHUNK, CHUNK)],
            wsems[b],
        ).wait()
    pltpu.async_copy(emb.at[iidxv.at[0]], buf0, gsem0)
    pltpu.make_async_copy(emb.at[iidxv.at[0]], buf0, gsem0).wait()
    pltpu.sync_copy(buf0, item_out.at[pl.ds(w * CHUNK, CHUNK)])


def _sc_gather(emb, sidx3, iidx3):
    f = functools.partial(
        pl.kernel,
        out_type=[
            jax.ShapeDtypeStruct((BATCH * N_SAMPLES, 2 * OUT), jnp.float32),
            jax.ShapeDtypeStruct((BATCH, 2 * OUT), jnp.float32),
        ],
        mesh=_mesh(),
        scratch_types=[
            pltpu.VMEM((_SRC_CH_W, CHUNK), jnp.int32),
            pltpu.VMEM((_ITEM_CH_W, CHUNK), jnp.int32),
            pltpu.VMEM((CHUNK, 2 * OUT), jnp.float32),
            pltpu.VMEM((CHUNK, 2 * OUT), jnp.float32),
            pltpu.SemaphoreType.DMA,
            pltpu.SemaphoreType.DMA,
            pltpu.SemaphoreType.DMA,
            pltpu.SemaphoreType.DMA,
        ],
    )(_sc_gather_body)
    return f(emb, sidx3, iidx3)


def kernel(x, edge_index, source, item_idx, W_self1, W_neigh1, b1,
           W_self2, W_neigh2, b2):
    src = edge_index[0].astype(jnp.int32)
    dst = edge_index[1].astype(jnp.int32)
    pad = E_PAD - N_EDGES
    srcg = jnp.concatenate([src, jnp.zeros((pad,), jnp.int32)]).reshape(
        E_PAD // (G * CHUNK), G, CHUNK
    )
    dstp = jnp.concatenate([dst, jnp.full((pad,), N_NODES, jnp.int32)])
    dstc = dstp.reshape(E_PAD // CHUNK, CHUNK)
    z128 = jnp.zeros((STRIPE, 128), jnp.float32)

    dstf = dstp.reshape(NW, E_PAD // NW)
    dg = _sc_deg(dstf)
    dga = dg[0, :N_NODES]
    dgb = dg[1, :N_NODES]

    s1, y1 = _tc1(x, W_self1, W_neigh1, b1)
    agg1 = _sc_layer(y1, srcg, dstc, z128)
    agg1a = agg1[0]
    agg1b = agg1[1]

    s2, y2p = _tc2(
        s1, agg1a[:N_NODES], agg1b[:N_NODES], dga, dgb, W_self2, W_neigh2, b2
    )
    agg2 = _sc_layer(y2p, srcg, dstc, z128)
    agg2a = agg2[0]
    agg2b = agg2[1]
    embp = _tc3(s2, agg2a[:N_NODES], agg2b[:N_NODES], dga, dgb)

    sidx3 = source.astype(jnp.int32).reshape(NW, _SRC_CH_W, CHUNK)
    iidx3 = item_idx.astype(jnp.int32).reshape(NW, _ITEM_CH_W, CHUNK)
    rated, item_emb = _sc_gather(embp, sidx3, iidx3)
    pred = _tc4(rated.reshape(BATCH, N_SAMPLES, 2 * OUT), item_emb)
    return pred
